# baseline (speedup 1.0000x reference)
import functools

import jax
import jax.numpy as jnp
import numpy as np
from jax import lax
from jax.experimental import pallas as pl
from jax.experimental.pallas import tpu as pltpu
from jax.experimental.pallas import tpu_sc as plsc

N_NODES = 10000
D_FEAT = 128
N_EDGES = 320000

NSUB = 16
FPS = D_FEAT // NSUB
ZSTRIDE = 5
ZWORDS = N_NODES * ZSTRIDE
B = 2560
NB = N_EDGES // B
ROWS = B // 16
PROWS = B // 128
LTILES = 10

_LN2_HI = np.float32(6.9313812256e-01)
_LN2_LO = np.float32(9.0580006145e-06)
_LG1 = np.float32(0.66666662693)
_LG2 = np.float32(0.40000972152)
_LG3 = np.float32(0.28498786688)
_LG4 = np.float32(0.24279078841)


def _softlog(y):
    yb = plsc.bitcast(y, jnp.int32)
    ix = yb + jnp.int32(0x3F800000 - 0x3F3504F3)
    e = lax.shift_right_logical(ix, jnp.int32(23)) - jnp.int32(127)
    mb = (ix & jnp.int32(0x007FFFFF)) + jnp.int32(0x3F3504F3)
    x = plsc.bitcast(mb, jnp.float32)
    f = x - 1.0
    s = f / (2.0 + f)
    z = s * s
    w = z * z
    t1 = w * (_LG2 + w * _LG4)
    t2 = z * (_LG1 + w * _LG3)
    r = t2 + t1
    hfsq = 0.5 * f * f
    dk = e.astype(jnp.float32)
    res = dk * _LN2_HI + ((f - hfsq) + (s * (hfsq + r) + dk * _LN2_LO))
    return jnp.where(y <= 0.0, jnp.float32(-jnp.inf), res)


def _make_sc_call():
    mesh = plsc.VectorSubcoreMesh(core_axis_name="c", subcore_axis_name="s")

    @functools.partial(
        pl.kernel,
        out_type=jax.ShapeDtypeStruct((2, NSUB, 16), jnp.float32),
        mesh=mesh,
        compiler_params=pltpu.CompilerParams(needs_layout_passes=False),
        scratch_types=[
            pltpu.VMEM((ZWORDS,), jnp.int32),
            pltpu.VMEM((B,), jnp.int32),
            pltpu.VMEM((B,), jnp.int32),
            pltpu.VMEM((PROWS, 128), jnp.float32),
            pltpu.VMEM((PROWS,), jnp.int32),
            pltpu.VMEM((PROWS, 128), jnp.float32),
            pltpu.VMEM((2, 128), jnp.float32),
            pltpu.VMEM((16,), jnp.float32),
            pltpu.VMEM_SHARED((PROWS, 128), jnp.float32),
        ],
    )
    def sc_loss(z2_hbm, src8_hbm, dst8_hbm, out_hbm,
                z_v, src_v, dst_v, part_v, iota_v, zero_v, dbuf_v, lout_v,
                acc_sh):
        c = lax.axis_index("c")
        s = lax.axis_index("s")

        pltpu.sync_copy(z2_hbm.at[s], z_v)

        lanes = lax.iota(jnp.int32, 16)
        iota_v[pl.ds(0, 16)] = lanes
        iota_v[pl.ds(PROWS - 16, 16)] = lanes + jnp.int32(PROWS - 16)
        zvec = jnp.zeros((16,), jnp.float32)
        for p in range(PROWS):
            for g in range(8):
                zero_v[p, pl.ds(g * 16, 16)] = zvec

        is_pos = c == 0
        avec = jnp.where(is_pos, jnp.float32(1.0), jnp.float32(-1.0)) + zvec
        bvec = jnp.where(is_pos, jnp.float32(1e-15), jnp.float32(1.0)) + zvec

        ebase = c * N_EDGES
        in_loss = s < LTILES
        lrow = jnp.where(in_loss, s * 2, 0)

        def batch_body(b_i, lacc):
            base = ebase + b_i * B
            pltpu.sync_copy(src8_hbm.at[pl.ds(base, B)], src_v)
            pltpu.sync_copy(dst8_hbm.at[pl.ds(base, B)], dst_v)

            def row_body(r):
                sv = src_v[pl.ds(r * 16, 16)]
                dv = dst_v[pl.ds(r * 16, 16)]
                hm = jnp.int32(-65536)
                sh = jnp.int32(16)
                acc = None
                for f in range(FPS // 2):
                    fo = jnp.int32(f)
                    aw = plsc.load_gather(z_v, [sv + fo])
                    bw = plsc.load_gather(z_v, [dv + fo])
                    alo = plsc.bitcast(lax.shift_left(aw, sh), jnp.float32)
                    blo = plsc.bitcast(lax.shift_left(bw, sh), jnp.float32)
                    ahi = plsc.bitcast(aw & hm, jnp.float32)
                    bhi = plsc.bitcast(bw & hm, jnp.float32)
                    t = alo * blo + ahi * bhi
                    acc = t if acc is None else acc + t
                rhi = lax.shift_right_logical(r, 3)
                rlo = (r & 7) * 16
                part_v[rhi, pl.ds(rlo, 16)] = acc

            plsc.parallel_loop(0, ROWS, 1, unroll=8)(row_body)

            plsc.subcore_barrier()

            @pl.when(s == 0)
            def _():
                pltpu.sync_copy(zero_v, acc_sh)

            plsc.subcore_barrier()
            pltpu.sync_copy(part_v, acc_sh.at[iota_v], add=True)
            plsc.subcore_barrier()

            pltpu.sync_copy(acc_sh.at[pl.ds(lrow, 2)], dbuf_v)
            for rr in range(2):
                for g in range(8):
                    d = dbuf_v[rr, pl.ds(g * 16, 16)]
                    u = jnp.exp(-d)
                    sg = 1.0 / (u + 1.0)
                    y = jnp.maximum(avec * sg + bvec, 0.0)
                    lg = _softlog(y)
                    lacc = lacc + jnp.where(in_loss, lg, 0.0)
            return lacc

        lacc = lax.fori_loop(0, NB, batch_body, jnp.zeros((16,), jnp.float32))
        lout_v[...] = lacc
        pltpu.sync_copy(lout_v, out_hbm.at[c, s])

    return sc_loss


_sc_loss = _make_sc_call()


def kernel(z, pos_edge_index, neg_edge_index):
    z = z.astype(jnp.float32)
    zb = z.astype(jnp.bfloat16).reshape(N_NODES, NSUB, FPS // 2, 2)
    zw = lax.bitcast_convert_type(zb, jnp.int32)
    z3 = zw.transpose(1, 0, 2)
    z3 = jnp.pad(z3, ((0, 0), (0, 0), (0, ZSTRIDE - FPS // 2)))
    z2 = z3.reshape(NSUB, ZWORDS)
    pe = pos_edge_index.astype(jnp.int32)
    ne = neg_edge_index.astype(jnp.int32)
    src8 = jnp.concatenate([pe[0], ne[0]]) * jnp.int32(ZSTRIDE)
    dst8 = jnp.concatenate([pe[1], ne[1]]) * jnp.int32(ZSTRIDE)
    parts = _sc_loss(z2, src8, dst8)
    return -jnp.sum(parts) / jnp.float32(N_EDGES)


# speedup vs baseline: 1.3549x; 1.3549x over previous
import functools

import jax
import jax.numpy as jnp
import numpy as np
from jax import lax
from jax.experimental import pallas as pl
from jax.experimental.pallas import tpu as pltpu
from jax.experimental.pallas import tpu_sc as plsc

N_NODES = 10000
D_FEAT = 128
N_EDGES = 320000

NSUB = 16
FPS = D_FEAT // NSUB
ZSTRIDE = 5
ZWORDS = N_NODES * ZSTRIDE
B = 2560
NB = N_EDGES // B
ROWS = B // 16
PROWS = B // 128
ACC_ROWS = NB * PROWS
ACC_PAD = NSUB * 8 * PROWS
LCH = 8

_LN2_HI = np.float32(6.9313812256e-01)
_LN2_LO = np.float32(9.0580006145e-06)
_LG1 = np.float32(0.66666662693)
_LG2 = np.float32(0.40000972152)
_LG3 = np.float32(0.28498786688)
_LG4 = np.float32(0.24279078841)


def _softlog(y):
    yb = plsc.bitcast(y, jnp.int32)
    ix = yb + jnp.int32(0x3F800000 - 0x3F3504F3)
    e = lax.shift_right_logical(ix, jnp.int32(23)) - jnp.int32(127)
    mb = (ix & jnp.int32(0x007FFFFF)) + jnp.int32(0x3F3504F3)
    x = plsc.bitcast(mb, jnp.float32)
    f = x - 1.0
    s = f / (2.0 + f)
    z = s * s
    w = z * z
    t1 = w * (_LG2 + w * _LG4)
    t2 = z * (_LG1 + w * _LG3)
    r = t2 + t1
    hfsq = 0.5 * f * f
    dk = e.astype(jnp.float32)
    res = dk * _LN2_HI + ((f - hfsq) + (s * (hfsq + r) + dk * _LN2_LO))
    return jnp.where(y <= 0.0, jnp.float32(-jnp.inf), res)


def _make_sc_call():
    mesh = plsc.VectorSubcoreMesh(core_axis_name="c", subcore_axis_name="s")

    @functools.partial(
        pl.kernel,
        out_type=jax.ShapeDtypeStruct((2, NSUB, 16), jnp.float32),
        mesh=mesh,
        compiler_params=pltpu.CompilerParams(needs_layout_passes=False),
        scratch_types=[
            pltpu.VMEM((ZWORDS,), jnp.int32),
            pltpu.VMEM((B,), jnp.int32),
            pltpu.VMEM((B,), jnp.int32),
            pltpu.VMEM((2 * PROWS, 128), jnp.float32),
            pltpu.VMEM((PROWS,), jnp.int32),
            pltpu.VMEM((PROWS,), jnp.int32),
            pltpu.VMEM((PROWS, 128), jnp.float32),
            pltpu.VMEM((16,), jnp.float32),
            pltpu.VMEM_SHARED((ACC_PAD, 128), jnp.float32),
            pltpu.SemaphoreType.DMA,
            pltpu.SemaphoreType.DMA,
        ],
    )
    def sc_loss(z2_hbm, src_hbm, dst_hbm, zeros_hbm, out_hbm,
                z_v, src_v, dst_v, part_v, iota_a, iota_b, dbuf_v, lout_v,
                acc_sh, sem_a, sem_b):
        c = lax.axis_index("c")
        s = lax.axis_index("s")

        pltpu.sync_copy(z2_hbm.at[s], z_v)

        for k in range(8):
            pltpu.sync_copy(zeros_hbm, acc_sh.at[pl.ds(s * 8 * PROWS + k * PROWS, PROWS)])

        lanes = lax.iota(jnp.int32, 16)
        zvec = jnp.zeros((16,), jnp.float32)

        is_pos = c == 0
        avec = jnp.where(is_pos, jnp.float32(1.0), jnp.float32(-1.0)) + zvec
        bvec = jnp.where(is_pos, jnp.float32(1e-15), jnp.float32(1.0)) + zvec

        ebase = c * N_EDGES
        soff = s * jnp.int32(8)

        plsc.subcore_barrier()

        def batch_body(b_i, carry):
            bp = lax.rem(b_i + soff, jnp.int32(NB))
            base = ebase + bp * B
            pltpu.sync_copy(src_hbm.at[pl.ds(base, B)], src_v)
            pltpu.sync_copy(dst_hbm.at[pl.ds(base, B)], dst_v)
            par = b_i & 1
            po = par * PROWS

            @pl.when(b_i >= 2)
            def _():
                @pl.when(par == 0)
                def _():
                    pltpu.make_async_copy(
                        zeros_hbm, part_v.at[pl.ds(0, PROWS)], sem_a).wait()

                @pl.when(par == 1)
                def _():
                    pltpu.make_async_copy(
                        zeros_hbm, part_v.at[pl.ds(PROWS, PROWS)], sem_b).wait()

            def row_body(r):
                sv = src_v[pl.ds(r * 16, 16)]
                dv = dst_v[pl.ds(r * 16, 16)]
                hm = jnp.int32(-65536)
                sh = jnp.int32(16)
                acc = None
                for f in range(FPS // 2):
                    fo = jnp.int32(f)
                    aw = plsc.load_gather(z_v, [sv + fo])
                    bw = plsc.load_gather(z_v, [dv + fo])
                    alo = plsc.bitcast(lax.shift_left(aw, sh), jnp.float32)
                    blo = plsc.bitcast(lax.shift_left(bw, sh), jnp.float32)
                    ahi = plsc.bitcast(aw & hm, jnp.float32)
                    bhi = plsc.bitcast(bw & hm, jnp.float32)
                    t = alo * blo + ahi * bhi
                    acc = t if acc is None else acc + t
                rhi = lax.shift_right_logical(r, 3) + po
                rlo = (r & 7) * 16
                part_v[rhi, pl.ds(rlo, 16)] = acc

            plsc.parallel_loop(0, ROWS, 1, unroll=8)(row_body)

            brow = bp * jnp.int32(PROWS)

            @pl.when(par == 0)
            def _():
                iota_a[pl.ds(0, 16)] = lanes + brow
                iota_a[pl.ds(PROWS - 16, 16)] = lanes + (brow + jnp.int32(PROWS - 16))
                pltpu.async_copy(part_v.at[pl.ds(0, PROWS)],
                                 acc_sh.at[iota_a], sem_a, add=True)

            @pl.when(par == 1)
            def _():
                iota_b[pl.ds(0, 16)] = lanes + brow
                iota_b[pl.ds(PROWS - 16, 16)] = lanes + (brow + jnp.int32(PROWS - 16))
                pltpu.async_copy(part_v.at[pl.ds(PROWS, PROWS)],
                                 acc_sh.at[iota_b], sem_b, add=True)

            return carry

        lax.fori_loop(0, NB, batch_body, jnp.int32(0))

        pltpu.make_async_copy(zeros_hbm, part_v.at[pl.ds(0, PROWS)], sem_a).wait()
        pltpu.make_async_copy(zeros_hbm, part_v.at[pl.ds(PROWS, PROWS)], sem_b).wait()
        plsc.subcore_barrier()

        start = s * jnp.int32(156) + jnp.minimum(s, jnp.int32(4))
        nrows = jnp.where(s < 4, jnp.int32(157), jnp.int32(156))

        def chunk_body(k, lacc_c):
            pltpu.sync_copy(acc_sh.at[pl.ds(start + k * PROWS, PROWS)], dbuf_v)

            def lrow_body(rr, lacc_r):
                valid = (k * PROWS + rr) < nrows
                contrib = zvec
                for g in range(8):
                    d = dbuf_v[rr, pl.ds(g * 16, 16)]
                    u = jnp.exp(-d)
                    sg = 1.0 / (u + 1.0)
                    y = jnp.maximum(avec * sg + bvec, 0.0)
                    contrib = contrib + _softlog(y)
                return lacc_r + jnp.where(valid, contrib, zvec)

            return lax.fori_loop(0, PROWS, lrow_body, lacc_c, unroll=2)

        lacc = lax.fori_loop(0, LCH, chunk_body, zvec)
        lout_v[...] = lacc
        pltpu.sync_copy(lout_v, out_hbm.at[c, s])

    return sc_loss


_sc_loss = _make_sc_call()


def kernel(z, pos_edge_index, neg_edge_index):
    z = z.astype(jnp.float32)
    zb = z.astype(jnp.bfloat16).reshape(N_NODES, NSUB, FPS // 2, 2)
    zw = lax.bitcast_convert_type(zb, jnp.int32)
    z3 = zw.transpose(1, 0, 2)
    z3 = jnp.pad(z3, ((0, 0), (0, 0), (0, ZSTRIDE - FPS // 2)))
    z2 = z3.reshape(NSUB, ZWORDS)
    pe = pos_edge_index.astype(jnp.int32)
    ne = neg_edge_index.astype(jnp.int32)
    src = jnp.concatenate([pe[0], ne[0]]) * jnp.int32(ZSTRIDE)
    dst = jnp.concatenate([pe[1], ne[1]]) * jnp.int32(ZSTRIDE)
    zeros = jnp.zeros((PROWS, 128), jnp.float32)
    parts = _sc_loss(z2, src, dst, zeros)
    return -jnp.sum(parts) / jnp.float32(N_EDGES)


# speedup vs baseline: 1.9408x; 1.4325x over previous
import functools

import jax
import jax.numpy as jnp
import numpy as np
from jax import lax
from jax.experimental import pallas as pl
from jax.experimental.pallas import tpu as pltpu
from jax.experimental.pallas import tpu_sc as plsc

N_NODES = 10000
D_FEAT = 128
N_EDGES = 320000

NSUB = 16
FPS = D_FEAT // NSUB
ZSTRIDE = 5
ZWORDS = N_NODES * ZSTRIDE
B = 2560
NB = N_EDGES // B
ROWS = B // 16
PROWS = B // 128
ACC_ROWS = NB * PROWS
ACC_PAD = NSUB * 8 * PROWS
LCH = 8

_LN2_HI = np.float32(6.9313812256e-01)
_LN2_LO = np.float32(9.0580006145e-06)
_LG1 = np.float32(0.66666662693)
_LG2 = np.float32(0.40000972152)
_LG3 = np.float32(0.28498786688)
_LG4 = np.float32(0.24279078841)


def _softlog(y):
    yb = plsc.bitcast(y, jnp.int32)
    ix = yb + jnp.int32(0x3F800000 - 0x3F3504F3)
    e = lax.shift_right_logical(ix, jnp.int32(23)) - jnp.int32(127)
    mb = (ix & jnp.int32(0x007FFFFF)) + jnp.int32(0x3F3504F3)
    x = plsc.bitcast(mb, jnp.float32)
    f = x - 1.0
    s = f / (2.0 + f)
    z = s * s
    w = z * z
    t1 = w * (_LG2 + w * _LG4)
    t2 = z * (_LG1 + w * _LG3)
    r = t2 + t1
    hfsq = 0.5 * f * f
    dk = e.astype(jnp.float32)
    res = dk * _LN2_HI + ((f - hfsq) + (s * (hfsq + r) + dk * _LN2_LO))
    return jnp.where(y <= 0.0, jnp.float32(-jnp.inf), res)


def _make_sc_call():
    mesh = plsc.VectorSubcoreMesh(core_axis_name="c", subcore_axis_name="s")

    @functools.partial(
        pl.kernel,
        out_type=jax.ShapeDtypeStruct((2, NSUB, 16), jnp.float32),
        mesh=mesh,
        compiler_params=pltpu.CompilerParams(needs_layout_passes=False),
        scratch_types=[
            pltpu.VMEM((ZWORDS,), jnp.int32),
            pltpu.VMEM((2 * B,), jnp.int32),
            pltpu.VMEM((2 * B,), jnp.int32),
            pltpu.VMEM((2 * PROWS, 128), jnp.float32),
            pltpu.VMEM((PROWS,), jnp.int32),
            pltpu.VMEM((PROWS,), jnp.int32),
            pltpu.VMEM((PROWS, 128), jnp.float32),
            pltpu.VMEM((16,), jnp.float32),
            pltpu.VMEM_SHARED((ACC_PAD, 128), jnp.float32),
            pltpu.SemaphoreType.DMA,
            pltpu.SemaphoreType.DMA,
            pltpu.SemaphoreType.DMA,
            pltpu.SemaphoreType.DMA,
        ],
    )
    def sc_loss(z2_hbm, src_hbm, dst_hbm, zeros_hbm, out_hbm,
                z_v, src_v, dst_v, part_v, iota_a, iota_b, dbuf_v, lout_v,
                acc_sh, sem_a, sem_b, sem_ia, sem_ib):
        c = lax.axis_index("c")
        s = lax.axis_index("s")

        pltpu.sync_copy(z2_hbm.at[s], z_v)

        for k in range(8):
            pltpu.sync_copy(zeros_hbm, acc_sh.at[pl.ds(s * 8 * PROWS + k * PROWS, PROWS)])

        lanes = lax.iota(jnp.int32, 16)
        zvec = jnp.zeros((16,), jnp.float32)

        is_pos = c == 0
        avec = jnp.where(is_pos, jnp.float32(1.0), jnp.float32(-1.0)) + zvec
        bvec = jnp.where(is_pos, jnp.float32(1e-15), jnp.float32(1.0)) + zvec

        ebase = c * N_EDGES
        soff = s * jnp.int32(8)

        plsc.subcore_barrier()

        base0 = ebase + lax.rem(soff, jnp.int32(NB)) * B
        pltpu.async_copy(src_hbm.at[pl.ds(base0, B)], src_v.at[pl.ds(0, B)], sem_ia)
        pltpu.async_copy(dst_hbm.at[pl.ds(base0, B)], dst_v.at[pl.ds(0, B)], sem_ia)

        def batch_body(b_i, carry):
            bp = lax.rem(b_i + soff, jnp.int32(NB))
            base = ebase + bp * B
            par = b_i & 1
            po = par * PROWS
            po2 = par * B

            @pl.when(b_i + 1 < NB)
            def _():
                bpn = lax.rem(b_i + 1 + soff, jnp.int32(NB))
                basen = ebase + bpn * B

                @pl.when(par == 0)
                def _():
                    pltpu.async_copy(src_hbm.at[pl.ds(basen, B)],
                                     src_v.at[pl.ds(B, B)], sem_ib)
                    pltpu.async_copy(dst_hbm.at[pl.ds(basen, B)],
                                     dst_v.at[pl.ds(B, B)], sem_ib)

                @pl.when(par == 1)
                def _():
                    pltpu.async_copy(src_hbm.at[pl.ds(basen, B)],
                                     src_v.at[pl.ds(0, B)], sem_ia)
                    pltpu.async_copy(dst_hbm.at[pl.ds(basen, B)],
                                     dst_v.at[pl.ds(0, B)], sem_ia)

            @pl.when(par == 0)
            def _():
                pltpu.make_async_copy(src_hbm.at[pl.ds(base, B)],
                                      src_v.at[pl.ds(0, B)], sem_ia).wait()
                pltpu.make_async_copy(dst_hbm.at[pl.ds(base, B)],
                                      dst_v.at[pl.ds(0, B)], sem_ia).wait()

            @pl.when(par == 1)
            def _():
                pltpu.make_async_copy(src_hbm.at[pl.ds(base, B)],
                                      src_v.at[pl.ds(B, B)], sem_ib).wait()
                pltpu.make_async_copy(dst_hbm.at[pl.ds(base, B)],
                                      dst_v.at[pl.ds(B, B)], sem_ib).wait()

            @pl.when(b_i >= 2)
            def _():
                @pl.when(par == 0)
                def _():
                    pltpu.make_async_copy(
                        zeros_hbm, part_v.at[pl.ds(0, PROWS)], sem_a).wait()

                @pl.when(par == 1)
                def _():
                    pltpu.make_async_copy(
                        zeros_hbm, part_v.at[pl.ds(PROWS, PROWS)], sem_b).wait()

            def row_body(r):
                sv = src_v[pl.ds(po2 + r * 16, 16)]
                dv = dst_v[pl.ds(po2 + r * 16, 16)]
                hm = jnp.int32(-65536)
                sh = jnp.int32(16)
                acc = None
                for f in range(FPS // 2):
                    fo = jnp.int32(f)
                    aw = plsc.load_gather(z_v, [sv + fo])
                    bw = plsc.load_gather(z_v, [dv + fo])
                    alo = plsc.bitcast(lax.shift_left(aw, sh), jnp.float32)
                    blo = plsc.bitcast(lax.shift_left(bw, sh), jnp.float32)
                    ahi = plsc.bitcast(aw & hm, jnp.float32)
                    bhi = plsc.bitcast(bw & hm, jnp.float32)
                    t = alo * blo + ahi * bhi
                    acc = t if acc is None else acc + t
                rhi = lax.shift_right_logical(r, 3) + po
                rlo = (r & 7) * 16
                part_v[rhi, pl.ds(rlo, 16)] = acc

            plsc.parallel_loop(0, ROWS, 1, unroll=8)(row_body)

            brow = bp * jnp.int32(PROWS)

            @pl.when(par == 0)
            def _():
                iota_a[pl.ds(0, 16)] = lanes + brow
                iota_a[pl.ds(PROWS - 16, 16)] = lanes + (brow + jnp.int32(PROWS - 16))
                pltpu.async_copy(part_v.at[pl.ds(0, PROWS)],
                                 acc_sh.at[iota_a], sem_a, add=True)

            @pl.when(par == 1)
            def _():
                iota_b[pl.ds(0, 16)] = lanes + brow
                iota_b[pl.ds(PROWS - 16, 16)] = lanes + (brow + jnp.int32(PROWS - 16))
                pltpu.async_copy(part_v.at[pl.ds(PROWS, PROWS)],
                                 acc_sh.at[iota_b], sem_b, add=True)

            return carry

        lax.fori_loop(0, NB, batch_body, jnp.int32(0))

        pltpu.make_async_copy(zeros_hbm, part_v.at[pl.ds(0, PROWS)], sem_a).wait()
        pltpu.make_async_copy(zeros_hbm, part_v.at[pl.ds(PROWS, PROWS)], sem_b).wait()
        plsc.subcore_barrier()

        start = s * jnp.int32(156) + jnp.minimum(s, jnp.int32(4))
        nrows = jnp.where(s < 4, jnp.int32(157), jnp.int32(156))

        def chunk_body(k, lacc_c):
            pltpu.sync_copy(acc_sh.at[pl.ds(start + k * PROWS, PROWS)], dbuf_v)

            def lrow_body(rr, lacc_r):
                valid = (k * PROWS + rr) < nrows
                contrib = zvec
                for g in range(8):
                    d = dbuf_v[rr, pl.ds(g * 16, 16)]
                    u = jnp.exp(-d)
                    sg = 1.0 / (u + 1.0)
                    y = jnp.maximum(avec * sg + bvec, 0.0)
                    contrib = contrib + _softlog(y)
                return lacc_r + jnp.where(valid, contrib, zvec)

            return lax.fori_loop(0, PROWS, lrow_body, lacc_c, unroll=2)

        lacc = lax.fori_loop(0, LCH, chunk_body, zvec)
        lout_v[...] = lacc
        pltpu.sync_copy(lout_v, out_hbm.at[c, s])

    return sc_loss


_sc_loss = _make_sc_call()


def kernel(z, pos_edge_index, neg_edge_index):
    z = z.astype(jnp.float32)
    zb = z.astype(jnp.bfloat16).reshape(N_NODES, NSUB, FPS // 2, 2)
    zw = lax.bitcast_convert_type(zb, jnp.int32)
    z3 = zw.transpose(1, 0, 2)
    z3 = jnp.pad(z3, ((0, 0), (0, 0), (0, ZSTRIDE - FPS // 2)))
    z2 = z3.reshape(NSUB, ZWORDS)
    pe = pos_edge_index.astype(jnp.int32)
    ne = neg_edge_index.astype(jnp.int32)
    src = jnp.concatenate([pe[0], ne[0]]) * jnp.int32(ZSTRIDE)
    dst = jnp.concatenate([pe[1], ne[1]]) * jnp.int32(ZSTRIDE)
    zeros = jnp.zeros((PROWS, 128), jnp.float32)
    parts = _sc_loss(z2, src, dst, zeros)
    return -jnp.sum(parts) / jnp.float32(N_EDGES)
